# baseline (speedup 1.0000x reference)
import functools

import jax
import jax.numpy as jnp
from jax.experimental import pallas as pl

B, N, K = 32, 2048, 20
M = B * N
EPS = 1e-5
BLK = 512

HI = jax.lax.Precision.HIGHEST
DEF = jax.lax.Precision.DEFAULT


def _lrelu(y):
    return jnp.where(y >= 0, y, 0.2 * y)


def _feat_y0(neigh_ref, x_ref, w_ref):
    nb = neigh_ref[...]
    ctr = x_ref[...][:, None, :]
    feat = jnp.concatenate(
        [nb - ctr, jnp.broadcast_to(ctr, nb.shape)], axis=-1)
    return jnp.dot(feat.reshape(BLK * K, 6), w_ref[...],
                   preferred_element_type=jnp.float32, precision=DEF)


def _s0_stats_body(neigh_ref, x_ref, w_ref, o_ref):
    i = pl.program_id(0)

    @pl.when(i == 0)
    def _init():
        o_ref[...] = jnp.zeros_like(o_ref)

    y0 = _feat_y0(neigh_ref, x_ref, w_ref)
    o_ref[0:1] += jnp.sum(y0, axis=0, keepdims=True)
    o_ref[1:2] += jnp.sum(y0 * y0, axis=0, keepdims=True)


def _s0_stats(neigh, x2d, w0):
    return pl.pallas_call(
        _s0_stats_body,
        grid=(M // BLK,),
        in_specs=[pl.BlockSpec((BLK, K, 3), lambda i: (i, 0, 0)),
                  pl.BlockSpec((BLK, 3), lambda i: (i, 0)),
                  pl.BlockSpec((6, 64), lambda i: (0, 0))],
        out_specs=pl.BlockSpec((2, 64), lambda i: (0, 0)),
        out_shape=jax.ShapeDtypeStruct((2, 64), jnp.float32),
    )(neigh, x2d, w0)


def _s0_apply_body(neigh_ref, x_ref, w_ref, st_ref, g_ref, b_ref, o_ref):
    st = st_ref[...]
    cnt = jnp.float32(M * K)
    mu = st[0:1] / cnt
    var = st[1:2] / cnt - mu * mu
    scale = g_ref[...] / jnp.sqrt(var + EPS)
    y0 = _feat_y0(neigh_ref, x_ref, w_ref).reshape(BLK, K, 64)
    ymax = jnp.max(y0, axis=1)
    o_ref[...] = _lrelu((ymax - mu) * scale + b_ref[...])


def _s0_apply(neigh, x2d, w0, st, g, b):
    return pl.pallas_call(
        _s0_apply_body,
        grid=(M // BLK,),
        in_specs=[pl.BlockSpec((BLK, K, 3), lambda i: (i, 0, 0)),
                  pl.BlockSpec((BLK, 3), lambda i: (i, 0)),
                  pl.BlockSpec((6, 64), lambda i: (0, 0)),
                  pl.BlockSpec((2, 64), lambda i: (0, 0)),
                  pl.BlockSpec((1, 64), lambda i: (0, 0)),
                  pl.BlockSpec((1, 64), lambda i: (0, 0))],
        out_specs=pl.BlockSpec((BLK, 64), lambda i: (i, 0)),
        out_shape=jax.ShapeDtypeStruct((M, 64), jnp.float32),
    )(neigh, x2d, w0, st, g, b)


def _stats_body(p_ref, sum_ref, gram_ref):
    i = pl.program_id(0)
    blk = p_ref[...]

    @pl.when(i == 0)
    def _init():
        sum_ref[...] = jnp.zeros_like(sum_ref)
        gram_ref[...] = jnp.zeros_like(gram_ref)

    sum_ref[...] += jnp.sum(blk, axis=0, keepdims=True)
    gram_ref[...] += jnp.dot(blk.T, blk,
                             preferred_element_type=jnp.float32, precision=HI)


def _stats(p):
    m, c = p.shape
    return pl.pallas_call(
        _stats_body,
        grid=(m // BLK,),
        in_specs=[pl.BlockSpec((BLK, c), lambda i: (i, 0))],
        out_specs=[pl.BlockSpec((1, c), lambda i: (0, 0)),
                   pl.BlockSpec((c, c), lambda i: (0, 0))],
        out_shape=[jax.ShapeDtypeStruct((1, c), jnp.float32),
                   jax.ShapeDtypeStruct((c, c), jnp.float32)],
    )(p)


def _finalize_body(w_ref, g_ref, sum_ref, gram_ref, mu_ref, sc_ref):
    w = w_ref[...]
    mean_p = sum_ref[...] / M
    mu = jnp.dot(mean_p, w, preferred_element_type=jnp.float32, precision=HI)
    gw = jnp.dot(gram_ref[...] / M, w,
                 preferred_element_type=jnp.float32, precision=HI)
    e2 = jnp.sum(w * gw, axis=0, keepdims=True)
    var = e2 - mu * mu
    mu_ref[...] = mu
    sc_ref[...] = g_ref[...] / jnp.sqrt(var + EPS)


def _finalize(w, g, sum_p, gram_p):
    cin, co = w.shape
    return pl.pallas_call(
        _finalize_body,
        in_specs=[pl.BlockSpec((cin, co), lambda: (0, 0)),
                  pl.BlockSpec((1, co), lambda: (0, 0)),
                  pl.BlockSpec((1, cin), lambda: (0, 0)),
                  pl.BlockSpec((cin, cin), lambda: (0, 0))],
        out_specs=[pl.BlockSpec((1, co), lambda: (0, 0)),
                   pl.BlockSpec((1, co), lambda: (0, 0))],
        out_shape=[jax.ShapeDtypeStruct((1, co), jnp.float32),
                   jax.ShapeDtypeStruct((1, co), jnp.float32)],
    )(w, g, sum_p, gram_p)


def _mm_bn_body(p_ref, w_ref, mu_ref, sc_ref, b_ref, o_ref):
    y = jnp.dot(p_ref[...], w_ref[...],
                preferred_element_type=jnp.float32, precision=DEF)
    o_ref[...] = _lrelu((y - mu_ref[...]) * sc_ref[...] + b_ref[...])


def _mm_bn(p, w, mu, sc, b):
    m, kc = p.shape
    co = w.shape[1]
    return pl.pallas_call(
        _mm_bn_body,
        grid=(m // BLK,),
        in_specs=[pl.BlockSpec((BLK, kc), lambda i: (i, 0)),
                  pl.BlockSpec((kc, co), lambda i: (0, 0)),
                  pl.BlockSpec((1, co), lambda i: (0, 0)),
                  pl.BlockSpec((1, co), lambda i: (0, 0)),
                  pl.BlockSpec((1, co), lambda i: (0, 0))],
        out_specs=pl.BlockSpec((BLK, co), lambda i: (i, 0)),
        out_shape=jax.ShapeDtypeStruct((m, co), jnp.float32),
    )(p, w, mu, sc, b)


def _final_body(h0_ref, h1_ref, h2_ref, h3_ref, w0_ref, w1_ref, w2_ref,
                w3_ref, b_ref, o_ref):
    j = pl.program_id(1)
    y = jnp.dot(h0_ref[0], w0_ref[...],
                preferred_element_type=jnp.float32, precision=DEF)
    y += jnp.dot(h1_ref[0], w1_ref[...],
                 preferred_element_type=jnp.float32, precision=DEF)
    y += jnp.dot(h2_ref[0], w2_ref[...],
                 preferred_element_type=jnp.float32, precision=DEF)
    y += jnp.dot(h3_ref[0], w3_ref[...],
                 preferred_element_type=jnp.float32, precision=DEF)
    blkmax = jnp.max(y, axis=0, keepdims=True)[None] + b_ref[...]

    @pl.when(j == 0)
    def _init():
        o_ref[...] = blkmax

    @pl.when(j > 0)
    def _acc():
        o_ref[...] = jnp.maximum(o_ref[...], blkmax)


def _final(h0, h1, h2, h3, wf, bf):
    cw = wf.shape[1]
    w0, w1, w2, w3 = wf[:64], wf[64:128], wf[128:256], wf[256:512]
    nb = N // BLK
    out = pl.pallas_call(
        _final_body,
        grid=(B, nb),
        in_specs=[
            pl.BlockSpec((1, BLK, 64), lambda b, j: (b, j, 0)),
            pl.BlockSpec((1, BLK, 64), lambda b, j: (b, j, 0)),
            pl.BlockSpec((1, BLK, 128), lambda b, j: (b, j, 0)),
            pl.BlockSpec((1, BLK, 256), lambda b, j: (b, j, 0)),
            pl.BlockSpec((64, cw), lambda b, j: (0, 0)),
            pl.BlockSpec((64, cw), lambda b, j: (0, 0)),
            pl.BlockSpec((128, cw), lambda b, j: (0, 0)),
            pl.BlockSpec((256, cw), lambda b, j: (0, 0)),
            pl.BlockSpec((1, 1, cw), lambda b, j: (0, 0, 0)),
        ],
        out_specs=pl.BlockSpec((1, 1, cw), lambda b, j: (b, 0, 0)),
        out_shape=jax.ShapeDtypeStruct((B, 1, cw), jnp.float32),
    )(h0.reshape(B, N, 64), h1.reshape(B, N, 64), h2.reshape(B, N, 128),
      h3.reshape(B, N, 256), w0, w1, w2, w3, bf.reshape(1, 1, cw))
    return out.reshape(B, cw)


def _pool_max(h, idx):
    g = jax.vmap(lambda t, i: t[i])(h, idx)
    return jnp.max(g, axis=2)


def _layer(p, w, g, b):
    sum_p, gram_p = _stats(p)
    mu, sc = _finalize(w, g.reshape(1, -1), sum_p, gram_p)
    return _mm_bn(p, w, mu, sc, b.reshape(1, -1))


def kernel(x, indices, W0, g0, b0, W1, g1, b1, W2, g2, b2, W3, g3, b3, Wf, bf):
    neigh = jax.vmap(lambda t, i: t[i])(x, indices)
    neigh = neigh.reshape(M, K, 3)
    x2d = x.reshape(M, 3)

    st = _s0_stats(neigh, x2d, W0)
    h0 = _s0_apply(neigh, x2d, W0, st, g0.reshape(1, -1), b0.reshape(1, -1))

    p1 = _pool_max(h0.reshape(B, N, 64), indices).reshape(M, 64)
    h1 = _layer(p1, W1, g1, b1)
    p2 = _pool_max(h1.reshape(B, N, 64), indices).reshape(M, 64)
    h2 = _layer(p2, W2, g2, b2)
    p3 = _pool_max(h2.reshape(B, N, 128), indices).reshape(M, 128)
    h3 = _layer(p3, W3, g3, b3)

    return _final(h0, h1, h2, h3, Wf, bf)


# speedup vs baseline: 26.7897x; 26.7897x over previous
import functools

import jax
import jax.numpy as jnp
from jax import lax
from jax.experimental import pallas as pl
from jax.experimental.pallas import tpu as pltpu
from jax.experimental.pallas import tpu_sc as plsc

B, N, K = 32, 2048, 20
M = B * N
EPS = 1e-5
BLK = 512

HI = jax.lax.Precision.HIGHEST
DEF = jax.lax.Precision.DEFAULT


def _lrelu(y):
    return jnp.where(y >= 0, y, 0.2 * y)


def _feat_y0(neigh_ref, x_ref, w_ref):
    nb = neigh_ref[...][:, :, :3]
    ctr = x_ref[...][:, None, :]
    feat = jnp.concatenate(
        [nb - ctr, jnp.broadcast_to(ctr, nb.shape)], axis=-1)
    return jnp.dot(feat.reshape(BLK * K, 6), w_ref[...],
                   preferred_element_type=jnp.float32, precision=DEF)


def _s0_stats_body(neigh_ref, x_ref, w_ref, o_ref):
    i = pl.program_id(0)

    @pl.when(i == 0)
    def _init():
        o_ref[...] = jnp.zeros_like(o_ref)

    y0 = _feat_y0(neigh_ref, x_ref, w_ref)
    o_ref[0:1] += jnp.sum(y0, axis=0, keepdims=True)
    o_ref[1:2] += jnp.sum(y0 * y0, axis=0, keepdims=True)


def _s0_stats(neigh, x2d, w0):
    return pl.pallas_call(
        _s0_stats_body,
        grid=(M // BLK,),
        in_specs=[pl.BlockSpec((BLK, K, 16), lambda i: (i, 0, 0)),
                  pl.BlockSpec((BLK, 3), lambda i: (i, 0)),
                  pl.BlockSpec((6, 64), lambda i: (0, 0))],
        out_specs=pl.BlockSpec((2, 64), lambda i: (0, 0)),
        out_shape=jax.ShapeDtypeStruct((2, 64), jnp.float32),
    )(neigh, x2d, w0)


def _s0_apply_body(neigh_ref, x_ref, w_ref, st_ref, g_ref, b_ref, o_ref):
    st = st_ref[...]
    cnt = jnp.float32(M * K)
    mu = st[0:1] / cnt
    var = st[1:2] / cnt - mu * mu
    scale = g_ref[...] / jnp.sqrt(var + EPS)
    y0 = _feat_y0(neigh_ref, x_ref, w_ref).reshape(BLK, K, 64)
    ymax = jnp.max(y0, axis=1)
    o_ref[...] = _lrelu((ymax - mu) * scale + b_ref[...])


def _s0_apply(neigh, x2d, w0, st, g, b):
    return pl.pallas_call(
        _s0_apply_body,
        grid=(M // BLK,),
        in_specs=[pl.BlockSpec((BLK, K, 16), lambda i: (i, 0, 0)),
                  pl.BlockSpec((BLK, 3), lambda i: (i, 0)),
                  pl.BlockSpec((6, 64), lambda i: (0, 0)),
                  pl.BlockSpec((2, 64), lambda i: (0, 0)),
                  pl.BlockSpec((1, 64), lambda i: (0, 0)),
                  pl.BlockSpec((1, 64), lambda i: (0, 0))],
        out_specs=pl.BlockSpec((BLK, 64), lambda i: (i, 0)),
        out_shape=jax.ShapeDtypeStruct((M, 64), jnp.float32),
    )(neigh, x2d, w0, st, g, b)


def _stats_body(p_ref, sum_ref, gram_ref):
    i = pl.program_id(0)
    blk = p_ref[...]

    @pl.when(i == 0)
    def _init():
        sum_ref[...] = jnp.zeros_like(sum_ref)
        gram_ref[...] = jnp.zeros_like(gram_ref)

    sum_ref[...] += jnp.sum(blk, axis=0, keepdims=True)
    gram_ref[...] += jnp.dot(blk.T, blk,
                             preferred_element_type=jnp.float32, precision=HI)


def _stats(p):
    m, c = p.shape
    return pl.pallas_call(
        _stats_body,
        grid=(m // BLK,),
        in_specs=[pl.BlockSpec((BLK, c), lambda i: (i, 0))],
        out_specs=[pl.BlockSpec((1, c), lambda i: (0, 0)),
                   pl.BlockSpec((c, c), lambda i: (0, 0))],
        out_shape=[jax.ShapeDtypeStruct((1, c), jnp.float32),
                   jax.ShapeDtypeStruct((c, c), jnp.float32)],
    )(p)


def _finalize_body(w_ref, g_ref, sum_ref, gram_ref, mu_ref, sc_ref):
    w = w_ref[...]
    mean_p = sum_ref[...] / M
    mu = jnp.dot(mean_p, w, preferred_element_type=jnp.float32, precision=HI)
    gw = jnp.dot(gram_ref[...] / M, w,
                 preferred_element_type=jnp.float32, precision=HI)
    e2 = jnp.sum(w * gw, axis=0, keepdims=True)
    var = e2 - mu * mu
    mu_ref[...] = mu
    sc_ref[...] = g_ref[...] / jnp.sqrt(var + EPS)


def _finalize(w, g, sum_p, gram_p):
    cin, co = w.shape
    return pl.pallas_call(
        _finalize_body,
        in_specs=[pl.BlockSpec((cin, co), lambda: (0, 0)),
                  pl.BlockSpec((1, co), lambda: (0, 0)),
                  pl.BlockSpec((1, cin), lambda: (0, 0)),
                  pl.BlockSpec((cin, cin), lambda: (0, 0))],
        out_specs=[pl.BlockSpec((1, co), lambda: (0, 0)),
                   pl.BlockSpec((1, co), lambda: (0, 0))],
        out_shape=[jax.ShapeDtypeStruct((1, co), jnp.float32),
                   jax.ShapeDtypeStruct((1, co), jnp.float32)],
    )(w, g, sum_p, gram_p)


def _mm_bn_body(p_ref, w_ref, mu_ref, sc_ref, b_ref, o_ref):
    y = jnp.dot(p_ref[...], w_ref[...],
                preferred_element_type=jnp.float32, precision=DEF)
    o_ref[...] = _lrelu((y - mu_ref[...]) * sc_ref[...] + b_ref[...])


def _mm_bn(p, w, mu, sc, b):
    m, kc = p.shape
    co = w.shape[1]
    return pl.pallas_call(
        _mm_bn_body,
        grid=(m // BLK,),
        in_specs=[pl.BlockSpec((BLK, kc), lambda i: (i, 0)),
                  pl.BlockSpec((kc, co), lambda i: (0, 0)),
                  pl.BlockSpec((1, co), lambda i: (0, 0)),
                  pl.BlockSpec((1, co), lambda i: (0, 0)),
                  pl.BlockSpec((1, co), lambda i: (0, 0))],
        out_specs=pl.BlockSpec((BLK, co), lambda i: (i, 0)),
        out_shape=jax.ShapeDtypeStruct((m, co), jnp.float32),
    )(p, w, mu, sc, b)


def _final_body(h0_ref, h1_ref, h2_ref, h3_ref, w0_ref, w1_ref, w2_ref,
                w3_ref, b_ref, o_ref):
    j = pl.program_id(1)
    y = jnp.dot(h0_ref[0], w0_ref[...],
                preferred_element_type=jnp.float32, precision=DEF)
    y += jnp.dot(h1_ref[0], w1_ref[...],
                 preferred_element_type=jnp.float32, precision=DEF)
    y += jnp.dot(h2_ref[0], w2_ref[...],
                 preferred_element_type=jnp.float32, precision=DEF)
    y += jnp.dot(h3_ref[0], w3_ref[...],
                 preferred_element_type=jnp.float32, precision=DEF)
    blkmax = jnp.max(y, axis=0, keepdims=True)[None] + b_ref[...]

    @pl.when(j == 0)
    def _init():
        o_ref[...] = blkmax

    @pl.when(j > 0)
    def _acc():
        o_ref[...] = jnp.maximum(o_ref[...], blkmax)


def _final(h0, h1, h2, h3, wf, bf):
    cw = wf.shape[1]
    w0, w1, w2, w3 = wf[:64], wf[64:128], wf[128:256], wf[256:512]
    nb = N // BLK
    out = pl.pallas_call(
        _final_body,
        grid=(B, nb),
        in_specs=[
            pl.BlockSpec((1, BLK, 64), lambda b, j: (b, j, 0)),
            pl.BlockSpec((1, BLK, 64), lambda b, j: (b, j, 0)),
            pl.BlockSpec((1, BLK, 128), lambda b, j: (b, j, 0)),
            pl.BlockSpec((1, BLK, 256), lambda b, j: (b, j, 0)),
            pl.BlockSpec((64, cw), lambda b, j: (0, 0)),
            pl.BlockSpec((64, cw), lambda b, j: (0, 0)),
            pl.BlockSpec((128, cw), lambda b, j: (0, 0)),
            pl.BlockSpec((256, cw), lambda b, j: (0, 0)),
            pl.BlockSpec((1, 1, cw), lambda b, j: (0, 0, 0)),
        ],
        out_specs=pl.BlockSpec((1, 1, cw), lambda b, j: (b, 0, 0)),
        out_shape=jax.ShapeDtypeStruct((B, 1, cw), jnp.float32),
    )(h0.reshape(B, N, 64), h1.reshape(B, N, 64), h2.reshape(B, N, 128),
      h3.reshape(B, N, 256), w0, w1, w2, w3, bf.reshape(1, 1, cw))
    return out.reshape(B, cw)


_SC_MESH = plsc.VectorSubcoreMesh(core_axis_name="c", subcore_axis_name="s")
_P = 32
_PAIRS = _P * K
_NSUB = _PAIRS // 128
_CHUNKS = N // _P


def _sc_pool_max(table, gidx2d, halves):
    def body(table_hbm, gidx_hbm, out_hbm, idx_v, idx2_v, rows_v, out_v, sem):
        wid = lax.axis_index("s") * 2 + lax.axis_index("c")

        for u in range(halves):
            def chunk(j, _):
                pltpu.sync_copy(gidx_hbm.at[wid * _CHUNKS + j], idx_v)
                if halves == 1:
                    src_idx = idx_v
                else:
                    for s in range(_NSUB):
                        for l in range(8):
                            v = idx_v[s, pl.ds(l * 16, 16)]
                            idx2_v[s, pl.ds(l * 16, 16)] = v * 2 + u
                    src_idx = idx2_v
                cps = [
                    pltpu.async_copy(
                        table_hbm.at[src_idx.at[s]],
                        rows_v.at[pl.ds(s * 128, 128)], sem)
                    for s in range(_NSUB)
                ]
                for cp in cps:
                    cp.wait()

                def point(p, _):
                    base = p * K
                    for g in range(4):
                        sl = pl.ds(g * 16, 16)
                        acc = rows_v[base, sl]
                        for k in range(1, K):
                            acc = jnp.maximum(acc, rows_v[base + k, sl])
                        out_v[p, sl] = acc
                    return 0

                lax.fori_loop(0, _P, point, 0)
                pltpu.sync_copy(out_v,
                                out_hbm.at[u, pl.ds(wid * N + j * _P, _P)])
                return 0

            lax.fori_loop(0, _CHUNKS, chunk, 0)

    f = pl.kernel(
        body,
        out_type=jax.ShapeDtypeStruct((halves, M, 64), jnp.float32),
        mesh=_SC_MESH,
        compiler_params=pltpu.CompilerParams(use_tc_tiling_on_sc=False),
        scratch_types=[
            pltpu.VMEM((_NSUB, 128), jnp.int32),
            pltpu.VMEM((_NSUB, 128), jnp.int32),
            pltpu.VMEM((_PAIRS, 64), jnp.float32),
            pltpu.VMEM((_P, 64), jnp.float32),
            pltpu.SemaphoreType.DMA,
        ],
    )
    return f(table, gidx2d)


def _sc_gather_x(x2d, gidx2d):
    def body(x_hbm, gidx_hbm, out_hbm, idx_v, rows_v, sem):
        wid = lax.axis_index("s") * 2 + lax.axis_index("c")

        def chunk(j, _):
            pltpu.sync_copy(gidx_hbm.at[wid * _CHUNKS + j], idx_v)
            cps = [
                pltpu.async_copy(x_hbm.at[idx_v.at[s]],
                                 rows_v.at[pl.ds(s * 128, 128)], sem)
                for s in range(_NSUB)
            ]
            for cp in cps:
                cp.wait()
            pltpu.sync_copy(
                rows_v, out_hbm.at[pl.ds((wid * N + j * _P) * K, _PAIRS)])
            return 0

        lax.fori_loop(0, _CHUNKS, chunk, 0)

    f = pl.kernel(
        body,
        out_type=jax.ShapeDtypeStruct((M * K, 16), jnp.float32),
        mesh=_SC_MESH,
        compiler_params=pltpu.CompilerParams(use_tc_tiling_on_sc=False),
        scratch_types=[
            pltpu.VMEM((_NSUB, 128), jnp.int32),
            pltpu.VMEM((_PAIRS, 16), jnp.float32),
            pltpu.SemaphoreType.DMA,
        ],
    )
    return f(x2d, gidx2d)


def _pool_max(h, gidx2d):
    m, c = h.shape
    halves = c // 64
    out = _sc_pool_max(h.reshape(halves * M, 64) if halves > 1 else h,
                       gidx2d, halves)
    if halves == 1:
        return out[0]
    return jnp.concatenate([out[0], out[1]], axis=-1)


def _layer(p, w, g, b):
    sum_p, gram_p = _stats(p)
    mu, sc = _finalize(w, g.reshape(1, -1), sum_p, gram_p)
    return _mm_bn(p, w, mu, sc, b.reshape(1, -1))


def kernel(x, indices, W0, g0, b0, W1, g1, b1, W2, g2, b2, W3, g3, b3, Wf, bf):
    x2d = x.reshape(M, 3)
    gidx = indices + (jnp.arange(B, dtype=jnp.int32) * N)[:, None, None]
    gidx2d = gidx.reshape(B * _CHUNKS, _NSUB, 128)

    x16 = jnp.pad(x2d, ((0, 0), (0, 13)))
    neigh = _sc_gather_x(x16, gidx2d).reshape(M, K, 16)

    st = _s0_stats(neigh, x2d, W0)
    h0 = _s0_apply(neigh, x2d, W0, st, g0.reshape(1, -1), b0.reshape(1, -1))

    p1 = _pool_max(h0, gidx2d)
    h1 = _layer(p1, W1, g1, b1)
    p2 = _pool_max(h1, gidx2d)
    h2 = _layer(p2, W2, g2, b2)
    p3 = _pool_max(h2, gidx2d)
    h3 = _layer(p3, W3, g3, b3)

    return _final(h0, h1, h2, h3, Wf, bf)


# speedup vs baseline: 34.6761x; 1.2944x over previous
import functools

import jax
import jax.numpy as jnp
from jax import lax
from jax.experimental import pallas as pl
from jax.experimental.pallas import tpu as pltpu
from jax.experimental.pallas import tpu_sc as plsc

B, N, K = 32, 2048, 20
M = B * N
EPS = 1e-5
BLK = 512

HI = jax.lax.Precision.HIGHEST
DEF = jax.lax.Precision.DEFAULT
F32 = jnp.float32
BF16 = jnp.bfloat16


def _lrelu(y):
    return jnp.where(y >= 0, y, 0.2 * y)


def _feat_y0(neigh_ref, x_ref, w_ref):
    nb = neigh_ref[...][:, :, :3]
    ctr = x_ref[...][:, None, :]
    feat = jnp.concatenate(
        [nb - ctr, jnp.broadcast_to(ctr, nb.shape)], axis=-1)
    return jnp.dot(feat.reshape(BLK * K, 6), w_ref[...],
                   preferred_element_type=F32, precision=DEF)


def _s0_stats_body(neigh_ref, x_ref, w_ref, o_ref):
    i = pl.program_id(0)

    @pl.when(i == 0)
    def _init():
        o_ref[...] = jnp.zeros_like(o_ref)

    y0 = _feat_y0(neigh_ref, x_ref, w_ref)
    o_ref[0:1] += jnp.sum(y0, axis=0, keepdims=True)
    o_ref[1:2] += jnp.sum(y0 * y0, axis=0, keepdims=True)


def _s0_stats(neigh, x2d, w0):
    return pl.pallas_call(
        _s0_stats_body,
        grid=(M // BLK,),
        in_specs=[pl.BlockSpec((BLK, K, 16), lambda i: (i, 0, 0)),
                  pl.BlockSpec((BLK, 3), lambda i: (i, 0)),
                  pl.BlockSpec((6, 64), lambda i: (0, 0))],
        out_specs=pl.BlockSpec((2, 64), lambda i: (0, 0)),
        out_shape=jax.ShapeDtypeStruct((2, 64), F32),
    )(neigh, x2d, w0)


def _s0_apply_body(neigh_ref, x_ref, w_ref, st_ref, g_ref, b_ref, o_ref):
    st = st_ref[...]
    cnt = jnp.float32(M * K)
    mu = st[0:1] / cnt
    var = st[1:2] / cnt - mu * mu
    scale = g_ref[...] / jnp.sqrt(var + EPS)
    y0 = _feat_y0(neigh_ref, x_ref, w_ref).reshape(BLK, K, 64)
    ymax = jnp.max(y0, axis=1)
    o_ref[...] = _lrelu((ymax - mu) * scale + b_ref[...]).astype(BF16)


def _s0_apply(neigh, x2d, w0, st, g, b):
    return pl.pallas_call(
        _s0_apply_body,
        grid=(M // BLK,),
        in_specs=[pl.BlockSpec((BLK, K, 16), lambda i: (i, 0, 0)),
                  pl.BlockSpec((BLK, 3), lambda i: (i, 0)),
                  pl.BlockSpec((6, 64), lambda i: (0, 0)),
                  pl.BlockSpec((2, 64), lambda i: (0, 0)),
                  pl.BlockSpec((1, 64), lambda i: (0, 0)),
                  pl.BlockSpec((1, 64), lambda i: (0, 0))],
        out_specs=pl.BlockSpec((BLK, 64), lambda i: (i, 0)),
        out_shape=jax.ShapeDtypeStruct((M, 64), BF16),
    )(neigh, x2d, w0, st, g, b)


def _stats_body(p_ref, sum_ref, gram_ref):
    i = pl.program_id(0)
    blk = p_ref[...].astype(F32)

    @pl.when(i == 0)
    def _init():
        sum_ref[...] = jnp.zeros_like(sum_ref)
        gram_ref[...] = jnp.zeros_like(gram_ref)

    sum_ref[...] += jnp.sum(blk, axis=0, keepdims=True)
    gram_ref[...] += jnp.dot(blk.T, blk,
                             preferred_element_type=F32, precision=HI)


def _stats(p):
    m, c = p.shape
    return pl.pallas_call(
        _stats_body,
        grid=(m // BLK,),
        in_specs=[pl.BlockSpec((BLK, c), lambda i: (i, 0))],
        out_specs=[pl.BlockSpec((1, c), lambda i: (0, 0)),
                   pl.BlockSpec((c, c), lambda i: (0, 0))],
        out_shape=[jax.ShapeDtypeStruct((1, c), F32),
                   jax.ShapeDtypeStruct((c, c), F32)],
    )(p)


def _mm_bn_body(p_ref, w_ref, g_ref, b_ref, sum_ref, gram_ref, o_ref):
    w = w_ref[...]
    mean_p = sum_ref[...] / M
    mu = jnp.dot(mean_p, w, preferred_element_type=F32, precision=HI)
    gw = jnp.dot(gram_ref[...] / M, w, preferred_element_type=F32,
                 precision=HI)
    e2 = jnp.sum(w * gw, axis=0, keepdims=True)
    var = e2 - mu * mu
    scale = g_ref[...] / jnp.sqrt(var + EPS)
    y = jnp.dot(p_ref[...].astype(F32), w,
                preferred_element_type=F32, precision=DEF)
    o_ref[...] = _lrelu((y - mu) * scale + b_ref[...]).astype(BF16)


def _mm_bn(p, w, g, b, sum_p, gram_p):
    m, kc = p.shape
    co = w.shape[1]
    return pl.pallas_call(
        _mm_bn_body,
        grid=(m // BLK,),
        in_specs=[pl.BlockSpec((BLK, kc), lambda i: (i, 0)),
                  pl.BlockSpec((kc, co), lambda i: (0, 0)),
                  pl.BlockSpec((1, co), lambda i: (0, 0)),
                  pl.BlockSpec((1, co), lambda i: (0, 0)),
                  pl.BlockSpec((1, kc), lambda i: (0, 0)),
                  pl.BlockSpec((kc, kc), lambda i: (0, 0))],
        out_specs=pl.BlockSpec((BLK, co), lambda i: (i, 0)),
        out_shape=jax.ShapeDtypeStruct((m, co), BF16),
    )(p, w, g, b, sum_p, gram_p)


def _final_body(h0_ref, h1_ref, h2_ref, h3_ref, w0_ref, w1_ref, w2_ref,
                w3_ref, b_ref, o_ref):
    j = pl.program_id(1)
    y = jnp.dot(h0_ref[0].astype(F32), w0_ref[...],
                preferred_element_type=F32, precision=DEF)
    y += jnp.dot(h1_ref[0].astype(F32), w1_ref[...],
                 preferred_element_type=F32, precision=DEF)
    y += jnp.dot(h2_ref[0].astype(F32), w2_ref[...],
                 preferred_element_type=F32, precision=DEF)
    y += jnp.dot(h3_ref[0].astype(F32), w3_ref[...],
                 preferred_element_type=F32, precision=DEF)
    blkmax = jnp.max(y, axis=0, keepdims=True)[None] + b_ref[...]

    @pl.when(j == 0)
    def _init():
        o_ref[...] = blkmax

    @pl.when(j > 0)
    def _acc():
        o_ref[...] = jnp.maximum(o_ref[...], blkmax)


def _final(h0, h1, h2, h3, wf, bf):
    cw = wf.shape[1]
    w0, w1, w2, w3 = wf[:64], wf[64:128], wf[128:256], wf[256:512]
    nb = N // BLK
    out = pl.pallas_call(
        _final_body,
        grid=(B, nb),
        in_specs=[
            pl.BlockSpec((1, BLK, 64), lambda b, j: (b, j, 0)),
            pl.BlockSpec((1, BLK, 64), lambda b, j: (b, j, 0)),
            pl.BlockSpec((1, BLK, 128), lambda b, j: (b, j, 0)),
            pl.BlockSpec((1, BLK, 256), lambda b, j: (b, j, 0)),
            pl.BlockSpec((64, cw), lambda b, j: (0, 0)),
            pl.BlockSpec((64, cw), lambda b, j: (0, 0)),
            pl.BlockSpec((128, cw), lambda b, j: (0, 0)),
            pl.BlockSpec((256, cw), lambda b, j: (0, 0)),
            pl.BlockSpec((1, 1, cw), lambda b, j: (0, 0, 0)),
        ],
        out_specs=pl.BlockSpec((1, 1, cw), lambda b, j: (b, 0, 0)),
        out_shape=jax.ShapeDtypeStruct((B, 1, cw), F32),
    )(h0.reshape(B, N, 64), h1.reshape(B, N, 64), h2.reshape(B, N, 128),
      h3.reshape(B, N, 256), w0, w1, w2, w3, bf.reshape(1, 1, cw))
    return out.reshape(B, cw)


_SC_MESH = plsc.VectorSubcoreMesh(core_axis_name="c", subcore_axis_name="s")
_PP = 64
_PPAIRS = _PP * K
_PNSUB = _PPAIRS // 128
_PCHUNKS = N // _PP
_XP = 128
_XPAIRS = _XP * K
_XNSUB = _XPAIRS // 128
_XCHUNKS = N // _XP


def _sc_pool_max(table, gidx3d, halves):
    def body(table_hbm, gidx_hbm, out_hbm, idx_a, idx_b, rows_a, rows_b,
             out_v, sem_a, sem_b):
        wid = lax.axis_index("s") * 2 + lax.axis_index("c")
        idx = (idx_a, idx_b)
        rows = (rows_a, rows_b)
        sems = (sem_a, sem_b)

        for u in range(halves):
            def load_issue(j, par):
                pltpu.sync_copy(gidx_hbm.at[wid * _PCHUNKS + j], idx[par])
                if halves == 2:
                    for s in range(_PNSUB):
                        for l in range(8):
                            sl = pl.ds(l * 16, 16)
                            idx[par][s, sl] = idx[par][s, sl] * 2 + u
                for s in range(_PNSUB):
                    pltpu.async_copy(table_hbm.at[idx[par].at[s]],
                                     rows[par].at[pl.ds(s * 128, 128)],
                                     sems[par])

            def consume(j, par):
                for s in range(_PNSUB):
                    pltpu.make_async_copy(
                        table_hbm.at[idx[par].at[s]],
                        rows[par].at[pl.ds(s * 128, 128)],
                        sems[par]).wait()

                def point(p, _):
                    base = p * K
                    for g in range(2):
                        sl = pl.ds(g * 32, 32)
                        acc = rows[par][base, sl]
                        for k in range(1, K):
                            acc = jnp.maximum(acc, rows[par][base + k, sl])
                        out_v[p, sl] = acc
                    return 0

                lax.fori_loop(0, _PP, point, 0)
                pltpu.sync_copy(out_v,
                                out_hbm.at[u, pl.ds(wid * N + j * _PP, _PP)])

            load_issue(0, 0)

            def pair(jj, _):
                j0 = 2 * jj
                load_issue(j0 + 1, 1)
                consume(j0, 0)

                @pl.when(jj < _PCHUNKS // 2 - 1)
                def _():
                    load_issue(j0 + 2, 0)

                consume(j0 + 1, 1)
                return 0

            lax.fori_loop(0, _PCHUNKS // 2, pair, 0)

    f = pl.kernel(
        body,
        out_type=jax.ShapeDtypeStruct((halves, M, 64), BF16),
        mesh=_SC_MESH,
        compiler_params=pltpu.CompilerParams(use_tc_tiling_on_sc=False),
        scratch_types=[
            pltpu.VMEM((_PNSUB, 128), jnp.int32),
            pltpu.VMEM((_PNSUB, 128), jnp.int32),
            pltpu.VMEM((_PPAIRS, 64), BF16),
            pltpu.VMEM((_PPAIRS, 64), BF16),
            pltpu.VMEM((_PP, 64), BF16),
            pltpu.SemaphoreType.DMA,
            pltpu.SemaphoreType.DMA,
        ],
    )
    return f(table, gidx3d)


def _sc_gather_x(x16, gidx3d):
    def body(x_hbm, gidx_hbm, out_hbm, idx_a, idx_b, rows_a, rows_b,
             sem_a, sem_b):
        wid = lax.axis_index("s") * 2 + lax.axis_index("c")
        idx = (idx_a, idx_b)
        rows = (rows_a, rows_b)
        sems = (sem_a, sem_b)

        def load_issue(j, par):
            pltpu.sync_copy(gidx_hbm.at[wid * _XCHUNKS + j], idx[par])
            for s in range(_XNSUB):
                pltpu.async_copy(x_hbm.at[idx[par].at[s]],
                                 rows[par].at[pl.ds(s * 128, 128)],
                                 sems[par])

        def consume(j, par):
            for s in range(_XNSUB):
                pltpu.make_async_copy(x_hbm.at[idx[par].at[s]],
                                      rows[par].at[pl.ds(s * 128, 128)],
                                      sems[par]).wait()
            pltpu.sync_copy(
                rows[par], out_hbm.at[pl.ds((wid * N + j * _XP) * K,
                                            _XPAIRS)])

        load_issue(0, 0)

        def pair(jj, _):
            j0 = 2 * jj
            load_issue(j0 + 1, 1)
            consume(j0, 0)

            @pl.when(jj < _XCHUNKS // 2 - 1)
            def _():
                load_issue(j0 + 2, 0)

            consume(j0 + 1, 1)
            return 0

        lax.fori_loop(0, _XCHUNKS // 2, pair, 0)

    f = pl.kernel(
        body,
        out_type=jax.ShapeDtypeStruct((M * K, 16), F32),
        mesh=_SC_MESH,
        compiler_params=pltpu.CompilerParams(use_tc_tiling_on_sc=False),
        scratch_types=[
            pltpu.VMEM((_XNSUB, 128), jnp.int32),
            pltpu.VMEM((_XNSUB, 128), jnp.int32),
            pltpu.VMEM((_XPAIRS, 16), F32),
            pltpu.VMEM((_XPAIRS, 16), F32),
            pltpu.SemaphoreType.DMA,
            pltpu.SemaphoreType.DMA,
        ],
    )
    return f(x16, gidx3d)


def _pool_max(h, gidx3d):
    m, c = h.shape
    halves = c // 64
    out = _sc_pool_max(h.reshape(halves * M, 64) if halves > 1 else h,
                       gidx3d, halves)
    if halves == 1:
        return out[0]
    return jnp.concatenate([out[0], out[1]], axis=-1)


def _layer(p, w, g, b):
    sum_p, gram_p = _stats(p)
    return _mm_bn(p, w, g.reshape(1, -1), b.reshape(1, -1), sum_p, gram_p)


def kernel(x, indices, W0, g0, b0, W1, g1, b1, W2, g2, b2, W3, g3, b3, Wf, bf):
    x2d = x.reshape(M, 3)
    gidx = indices + (jnp.arange(B, dtype=jnp.int32) * N)[:, None, None]
    gidx_flat = gidx.reshape(-1)
    gidx_pool = gidx_flat.reshape(B * _PCHUNKS, _PNSUB, 128)
    gidx_x = gidx_flat.reshape(B * _XCHUNKS, _XNSUB, 128)

    x16 = jnp.pad(x2d, ((0, 0), (0, 13)))
    neigh = _sc_gather_x(x16, gidx_x).reshape(M, K, 16)

    st = _s0_stats(neigh, x2d, W0)
    h0 = _s0_apply(neigh, x2d, W0, st, g0.reshape(1, -1), b0.reshape(1, -1))

    p1 = _pool_max(h0, gidx_pool)
    h1 = _layer(p1, W1, g1, b1)
    p2 = _pool_max(h1, gidx_pool)
    h2 = _layer(p2, W2, g2, b2)
    p3 = _pool_max(h2, gidx_pool)
    h3 = _layer(p3, W3, g3, b3)

    return _final(h0, h1, h2, h3, Wf, bf)


# speedup vs baseline: 44.5946x; 1.2860x over previous
import functools

import jax
import jax.numpy as jnp
from jax import lax
from jax.experimental import pallas as pl
from jax.experimental.pallas import tpu as pltpu
from jax.experimental.pallas import tpu_sc as plsc
from jax.scipy.linalg import block_diag

B, N, K = 32, 2048, 20
M = B * N
EPS = 1e-5
BLK = 512

HI = jax.lax.Precision.HIGHEST
DEF = jax.lax.Precision.DEFAULT
F32 = jnp.float32
BF16 = jnp.bfloat16


def _lrelu(y):
    return jnp.where(y >= 0, y, 0.2 * y)


_PKB = BLK * K // 8


def _pack_y0(pk_ref, w_ref):
    return jnp.dot(pk_ref[...], w_ref[...],
                   preferred_element_type=F32, precision=DEF)


def _s0_stats_body(pk_ref, w_ref, o_ref):
    i = pl.program_id(0)

    @pl.when(i == 0)
    def _init():
        o_ref[...] = jnp.zeros_like(o_ref)

    y = _pack_y0(pk_ref, w_ref)
    s1 = jnp.sum(y, axis=0, keepdims=True)
    s2 = jnp.sum(y * y, axis=0, keepdims=True)
    o_ref[0:1] += sum(s1[:, 64 * s:64 * s + 64] for s in range(8))
    o_ref[1:2] += sum(s2[:, 64 * s:64 * s + 64] for s in range(8))


def _s0_stats(featpack, w0stack):
    return pl.pallas_call(
        _s0_stats_body,
        grid=(M // BLK,),
        in_specs=[pl.BlockSpec((_PKB, 128), lambda i: (i, 0)),
                  pl.BlockSpec((128, 512), lambda i: (0, 0))],
        out_specs=pl.BlockSpec((2, 64), lambda i: (0, 0)),
        out_shape=jax.ShapeDtypeStruct((2, 64), F32),
    )(featpack, w0stack)


def _s0_apply_body(pk_ref, w_ref, st_ref, g_ref, b_ref, o_ref):
    st = st_ref[...]
    cnt = jnp.float32(M * K)
    mu = st[0:1] / cnt
    var = st[1:2] / cnt - mu * mu
    scale = g_ref[...] / jnp.sqrt(var + EPS)
    y = _pack_y0(pk_ref, w_ref)
    ys = [y[:, 64 * s:64 * s + 64] for s in range(8)]
    full = ys[0]
    for s in range(1, 8):
        full = jnp.maximum(full, ys[s])
    low = jnp.maximum(jnp.maximum(ys[0], ys[1]), jnp.maximum(ys[2], ys[3]))
    high = jnp.maximum(jnp.maximum(ys[4], ys[5]), jnp.maximum(ys[6], ys[7]))
    r5 = full.reshape(_PKB // 5, 5, 64)
    l5 = low.reshape(_PKB // 5, 5, 64)
    h5 = high.reshape(_PKB // 5, 5, 64)
    even = jnp.maximum(jnp.maximum(r5[:, 0], r5[:, 1]), l5[:, 2])
    odd = jnp.maximum(jnp.maximum(h5[:, 2], r5[:, 3]), r5[:, 4])
    ymax = jnp.stack([even, odd], axis=1).reshape(BLK, 64)
    o_ref[...] = _lrelu((ymax - mu) * scale + b_ref[...]).astype(BF16)


def _s0_apply(featpack, w0stack, st, g, b):
    return pl.pallas_call(
        _s0_apply_body,
        grid=(M // BLK,),
        in_specs=[pl.BlockSpec((_PKB, 128), lambda i: (i, 0)),
                  pl.BlockSpec((128, 512), lambda i: (0, 0)),
                  pl.BlockSpec((2, 64), lambda i: (0, 0)),
                  pl.BlockSpec((1, 64), lambda i: (0, 0)),
                  pl.BlockSpec((1, 64), lambda i: (0, 0))],
        out_specs=pl.BlockSpec((BLK, 64), lambda i: (i, 0)),
        out_shape=jax.ShapeDtypeStruct((M, 64), BF16),
    )(featpack, w0stack, st, g, b)


def _stats_body(p_ref, sum_ref, gram_ref):
    i = pl.program_id(0)
    blk = p_ref[...].astype(F32)

    @pl.when(i == 0)
    def _init():
        sum_ref[...] = jnp.zeros_like(sum_ref)
        gram_ref[...] = jnp.zeros_like(gram_ref)

    sum_ref[...] += jnp.sum(blk, axis=0, keepdims=True)
    gram_ref[...] += jnp.dot(blk.T, blk,
                             preferred_element_type=F32, precision=HI)


def _stats(p):
    m, c = p.shape
    return pl.pallas_call(
        _stats_body,
        grid=(m // BLK,),
        in_specs=[pl.BlockSpec((BLK, c), lambda i: (i, 0))],
        out_specs=[pl.BlockSpec((1, c), lambda i: (0, 0)),
                   pl.BlockSpec((c, c), lambda i: (0, 0))],
        out_shape=[jax.ShapeDtypeStruct((1, c), F32),
                   jax.ShapeDtypeStruct((c, c), F32)],
    )(p)


def _mm_bn_body(p_ref, w_ref, g_ref, b_ref, sum_ref, gram_ref, o_ref):
    w = w_ref[...]
    mean_p = sum_ref[...] / M
    mu = jnp.dot(mean_p, w, preferred_element_type=F32, precision=HI)
    gw = jnp.dot(gram_ref[...] / M, w, preferred_element_type=F32,
                 precision=HI)
    e2 = jnp.sum(w * gw, axis=0, keepdims=True)
    var = e2 - mu * mu
    scale = g_ref[...] / jnp.sqrt(var + EPS)
    y = jnp.dot(p_ref[...].astype(F32), w,
                preferred_element_type=F32, precision=DEF)
    o_ref[...] = _lrelu((y - mu) * scale + b_ref[...]).astype(BF16)


def _mm_bn(p, w, g, b, sum_p, gram_p):
    m, kc = p.shape
    co = w.shape[1]
    return pl.pallas_call(
        _mm_bn_body,
        grid=(m // BLK,),
        in_specs=[pl.BlockSpec((BLK, kc), lambda i: (i, 0)),
                  pl.BlockSpec((kc, co), lambda i: (0, 0)),
                  pl.BlockSpec((1, co), lambda i: (0, 0)),
                  pl.BlockSpec((1, co), lambda i: (0, 0)),
                  pl.BlockSpec((1, kc), lambda i: (0, 0)),
                  pl.BlockSpec((kc, kc), lambda i: (0, 0))],
        out_specs=pl.BlockSpec((BLK, co), lambda i: (i, 0)),
        out_shape=jax.ShapeDtypeStruct((m, co), BF16),
    )(p, w, g, b, sum_p, gram_p)


def _final_body(h0_ref, h1_ref, h2_ref, h3_ref, w0_ref, w1_ref, w2_ref,
                w3_ref, b_ref, o_ref):
    j = pl.program_id(1)
    y = jnp.dot(h0_ref[0].astype(F32), w0_ref[...],
                preferred_element_type=F32, precision=DEF)
    y += jnp.dot(h1_ref[0].astype(F32), w1_ref[...],
                 preferred_element_type=F32, precision=DEF)
    y += jnp.dot(h2_ref[0].astype(F32), w2_ref[...],
                 preferred_element_type=F32, precision=DEF)
    y += jnp.dot(h3_ref[0].astype(F32), w3_ref[...],
                 preferred_element_type=F32, precision=DEF)
    blkmax = jnp.max(y, axis=0, keepdims=True)[None] + b_ref[...]

    @pl.when(j == 0)
    def _init():
        o_ref[...] = blkmax

    @pl.when(j > 0)
    def _acc():
        o_ref[...] = jnp.maximum(o_ref[...], blkmax)


def _final(h0, h1, h2, h3, wf, bf):
    cw = wf.shape[1]
    w0, w1, w2, w3 = wf[:64], wf[64:128], wf[128:256], wf[256:512]
    nb = N // BLK
    out = pl.pallas_call(
        _final_body,
        grid=(B, nb),
        in_specs=[
            pl.BlockSpec((1, BLK, 64), lambda b, j: (b, j, 0)),
            pl.BlockSpec((1, BLK, 64), lambda b, j: (b, j, 0)),
            pl.BlockSpec((1, BLK, 128), lambda b, j: (b, j, 0)),
            pl.BlockSpec((1, BLK, 256), lambda b, j: (b, j, 0)),
            pl.BlockSpec((64, cw), lambda b, j: (0, 0)),
            pl.BlockSpec((64, cw), lambda b, j: (0, 0)),
            pl.BlockSpec((128, cw), lambda b, j: (0, 0)),
            pl.BlockSpec((256, cw), lambda b, j: (0, 0)),
            pl.BlockSpec((1, 1, cw), lambda b, j: (0, 0, 0)),
        ],
        out_specs=pl.BlockSpec((1, 1, cw), lambda b, j: (b, 0, 0)),
        out_shape=jax.ShapeDtypeStruct((B, 1, cw), F32),
    )(h0.reshape(B, N, 64), h1.reshape(B, N, 64), h2.reshape(B, N, 128),
      h3.reshape(B, N, 256), w0, w1, w2, w3, bf.reshape(1, 1, cw))
    return out.reshape(B, cw)


_SC_MESH = plsc.VectorSubcoreMesh(core_axis_name="c", subcore_axis_name="s")
_PP = 64
_PPAIRS = _PP * K
_PNSUB = _PPAIRS // 128
_PCHUNKS = N // _PP
_XP = 64
_XPAIRS = _XP * K
_XNSUB = _XPAIRS // 128
_XCHUNKS = N // _XP
_XOUT = _XPAIRS // 8


def _sc_pool_max(table, gidx3d, halves):
    def body(table_hbm, gidx_hbm, out_hbm, idx_a, idx_b, rows_a, rows_b,
             out_v, sem_a, sem_b):
        wid = lax.axis_index("s") * 2 + lax.axis_index("c")
        idx = (idx_a, idx_b)
        rows = (rows_a, rows_b)
        sems = (sem_a, sem_b)

        for u in range(halves):
            def load_issue(j, par):
                pltpu.sync_copy(gidx_hbm.at[wid * _PCHUNKS + j], idx[par])
                if halves == 2:
                    for s in range(_PNSUB):
                        for l in range(8):
                            sl = pl.ds(l * 16, 16)
                            idx[par][s, sl] = idx[par][s, sl] * 2 + u
                for s in range(_PNSUB):
                    pltpu.async_copy(table_hbm.at[idx[par].at[s]],
                                     rows[par].at[pl.ds(s * 128, 128)],
                                     sems[par])

            def consume(j, par):
                for s in range(_PNSUB):
                    pltpu.make_async_copy(
                        table_hbm.at[idx[par].at[s]],
                        rows[par].at[pl.ds(s * 128, 128)],
                        sems[par]).wait()

                def point(p, _):
                    base = p * K
                    for g in range(2):
                        sl = pl.ds(g * 32, 32)
                        acc = rows[par][base, sl]
                        for k in range(1, K):
                            acc = jnp.maximum(acc, rows[par][base + k, sl])
                        out_v[p, sl] = acc
                    return 0

                lax.fori_loop(0, _PP, point, 0)
                pltpu.sync_copy(out_v,
                                out_hbm.at[u, pl.ds(wid * N + j * _PP, _PP)])

            load_issue(0, 0)

            def pair(jj, _):
                j0 = 2 * jj
                load_issue(j0 + 1, 1)
                consume(j0, 0)

                @pl.when(jj < _PCHUNKS // 2 - 1)
                def _():
                    load_issue(j0 + 2, 0)

                consume(j0 + 1, 1)
                return 0

            lax.fori_loop(0, _PCHUNKS // 2, pair, 0)

    f = pl.kernel(
        body,
        out_type=jax.ShapeDtypeStruct((halves, M, 64), BF16),
        mesh=_SC_MESH,
        compiler_params=pltpu.CompilerParams(use_tc_tiling_on_sc=False),
        scratch_types=[
            pltpu.VMEM((_PNSUB, 128), jnp.int32),
            pltpu.VMEM((_PNSUB, 128), jnp.int32),
            pltpu.VMEM((_PPAIRS, 64), BF16),
            pltpu.VMEM((_PPAIRS, 64), BF16),
            pltpu.VMEM((_PP, 64), BF16),
            pltpu.SemaphoreType.DMA,
            pltpu.SemaphoreType.DMA,
        ],
    )
    return f(table, gidx3d)


def _sc_edge_feat(xt, gidx3d):
    def body(xt_hbm, gidx_hbm, out_hbm, idx_a, idx_b, rows_a, rows_b,
             outp_a, outp_b, ctr_v, sem_a, sem_b):
        wid = lax.axis_index("s") * 2 + lax.axis_index("c")
        idx = (idx_a, idx_b)
        rows = (rows_a, rows_b)
        outp = (outp_a, outp_b)
        sems = (sem_a, sem_b)
        lanes = lax.iota(jnp.int32, 16)
        dmask = lanes < 3

        def load_issue(j, par):
            pltpu.sync_copy(gidx_hbm.at[wid * _XCHUNKS + j], idx[par])
            for s in range(_XNSUB):
                pltpu.async_copy(xt_hbm.at[idx[par].at[s]],
                                 rows[par].at[pl.ds(s * 128, 128)],
                                 sems[par])

        def consume(j, par):
            pltpu.sync_copy(xt_hbm.at[pl.ds(wid * N + j * _XP, _XP)], ctr_v)
            for s in range(_XNSUB):
                pltpu.make_async_copy(xt_hbm.at[idx[par].at[s]],
                                      rows[par].at[pl.ds(s * 128, 128)],
                                      sems[par]).wait()

            def point(n, _):
                ctr = ctr_v[n, :]
                for k in range(K):
                    q = n * K + k
                    d = rows[par][q, :] - ctr
                    feat = jnp.where(dmask, d, ctr)
                    outp[par][q // 8, pl.ds((q % 8) * 16, 16)] = feat
                return 0

            lax.fori_loop(0, _XP, point, 0)
            pltpu.sync_copy(outp[par],
                            out_hbm.at[pl.ds((wid * N + j * _XP) * K // 8,
                                             _XOUT)])

        load_issue(0, 0)

        def pair(jj, _):
            j0 = 2 * jj
            load_issue(j0 + 1, 1)
            consume(j0, 0)

            @pl.when(jj < _XCHUNKS // 2 - 1)
            def _():
                load_issue(j0 + 2, 0)

            consume(j0 + 1, 1)
            return 0

        lax.fori_loop(0, _XCHUNKS // 2, pair, 0)

    f = pl.kernel(
        body,
        out_type=jax.ShapeDtypeStruct((M * K // 8, 128), F32),
        mesh=_SC_MESH,
        compiler_params=pltpu.CompilerParams(use_tc_tiling_on_sc=False),
        scratch_types=[
            pltpu.VMEM((_XNSUB, 128), jnp.int32),
            pltpu.VMEM((_XNSUB, 128), jnp.int32),
            pltpu.VMEM((_XPAIRS, 16), F32),
            pltpu.VMEM((_XPAIRS, 16), F32),
            pltpu.VMEM((_XOUT, 128), F32),
            pltpu.VMEM((_XOUT, 128), F32),
            pltpu.VMEM((_XP, 16), F32),
            pltpu.SemaphoreType.DMA,
            pltpu.SemaphoreType.DMA,
        ],
    )
    return f(xt, gidx3d)


def _pool_max(h, gidx3d):
    m, c = h.shape
    halves = c // 64
    out = _sc_pool_max(h.reshape(halves * M, 64) if halves > 1 else h,
                       gidx3d, halves)
    if halves == 1:
        return out[0]
    return jnp.concatenate([out[0], out[1]], axis=-1)


def _layer(p, w, g, b):
    sum_p, gram_p = _stats(p)
    return _mm_bn(p, w, g.reshape(1, -1), b.reshape(1, -1), sum_p, gram_p)


def kernel(x, indices, W0, g0, b0, W1, g1, b1, W2, g2, b2, W3, g3, b3, Wf, bf):
    x2d = x.reshape(M, 3)
    gidx = indices + (jnp.arange(B, dtype=jnp.int32) * N)[:, None, None]
    gidx_flat = gidx.reshape(-1)
    gidx_pool = gidx_flat.reshape(B * _PCHUNKS, _PNSUB, 128)
    gidx_x = gidx_flat.reshape(B * _XCHUNKS, _XNSUB, 128)

    xt = jnp.concatenate(
        [x2d, x2d, jnp.zeros((M, 10), F32)], axis=1)
    featpack = _sc_edge_feat(xt, gidx_x)

    w0p = jnp.pad(W0, ((0, 10), (0, 0)))
    w0stack = block_diag(*([w0p] * 8))
    st = _s0_stats(featpack, w0stack)
    h0 = _s0_apply(featpack, w0stack, st,
                   g0.reshape(1, -1), b0.reshape(1, -1))

    p1 = _pool_max(h0, gidx_pool)
    h1 = _layer(p1, W1, g1, b1)
    p2 = _pool_max(h1, gidx_pool)
    h2 = _layer(p2, W2, g2, b2)
    p3 = _pool_max(h2, gidx_pool)
    h3 = _layer(p3, W3, g3, b3)

    return _final(h0, h1, h2, h3, Wf, bf)
